# baseline (speedup 1.0000x reference)
import functools

import jax
import jax.numpy as jnp
from jax import lax
from jax.experimental import pallas as pl
from jax.experimental.pallas import tpu as pltpu
from jax.experimental.pallas import tpu_sc as plsc

EMBED = 768
DEPTH = 128
LANES = 16

_NW = 32
_CK = 4
_NS = 8
_LA = 6


def _rne_to_int(x):
    s = x + 0.5
    t = s.astype(jnp.int32)
    d = s - t.astype(jnp.float32)
    nonint = jnp.minimum((d * 16777216.0).astype(jnp.int32), 1)
    return t - ((1 - nonint) & t & 1)


def _mean_idx(f, t):
    s = f + t
    h = s >> 1
    i = h + ((s & h) & 1)
    return jnp.minimum(jnp.maximum(i, 0), DEPTH - 1)


def _body(tpw, in_hbm, rpf_hbm, rpt_hbm, cpf_hbm, cpt_hbm, tab_hbm,
          out_hbm, tabl, posb, ridx, cidx, sbase, *slotrefs):
    inb = slotrefs[0:_NS]
    semg = slotrefs[_NS:2 * _NS]
    semo = slotrefs[2 * _NS:3 * _NS]

    wid = lax.axis_index("s") * 2 + lax.axis_index("c")
    base = wid * tpw
    nc = tpw // _CK

    pltpu.sync_copy(tab_hbm, tabl)
    pltpu.sync_copy(rpf_hbm.at[pl.ds(base, tpw)], posb.at[0])
    pltpu.sync_copy(rpt_hbm.at[pl.ds(base, tpw)], posb.at[1])
    pltpu.sync_copy(cpf_hbm.at[pl.ds(base, tpw)], posb.at[2])
    pltpu.sync_copy(cpt_hbm.at[pl.ds(base, tpw)], posb.at[3])

    def idx_body(g, carry):
        sl = pl.ds(g * LANES, LANES)
        rf = _rne_to_int(posb[0, sl] * float(DEPTH))
        rt = _rne_to_int(posb[1, sl] * float(DEPTH))
        cf = _rne_to_int(posb[2, sl] * float(DEPTH))
        ct = _rne_to_int(posb[3, sl] * float(DEPTH))
        ridx[sl] = _mean_idx(rf, rt)
        cidx[sl] = _mean_idx(cf, ct) + DEPTH
        return carry

    lax.fori_loop(0, tpw // LANES, idx_body, 0)

    def issue(cc, s):
        t0 = base + cc * _CK
        pltpu.async_copy(in_hbm.at[pl.ds(t0, _CK)], inb[s], semg[s])

    def drain_out(s):
        pltpu.make_async_copy(inb[s], out_hbm.at[pl.ds(base, _CK)],
                              semo[s]).wait()

    def compute(cc, s, sub):
        t0 = base + cc * _CK
        src = in_hbm.at[pl.ds(t0, _CK)]
        pltpu.make_async_copy(src, inb[s], semg[s]).wait()
        pb = (cc - sub) * _CK
        rvec = ridx[pl.ds(pb, LANES)] * (EMBED // 2)
        cvec = cidx[pl.ds(pb, LANES)] * (EMBED // 2)
        for l in range(_CK):
            sbase[l] = rvec[sub * _CK + l]
            sbase[_CK + l] = cvec[sub * _CK + l]

        def dim_group(d2, carry):
            for u in range(2):
                d = d2 * 2 + u
                doff = d * LANES
                sla = pl.ds(d * 2 * LANES, LANES)
                slb = pl.ds(d * 2 * LANES + LANES, LANES)
                for l in range(_CK):
                    rw = tabl[pl.ds(sbase[l] + doff, LANES)]
                    cw = tabl[pl.ds(sbase[_CK + l] + doff, LANES)]
                    ra = lax.bitcast_convert_type(rw << 16, jnp.float32)
                    rb = lax.bitcast_convert_type(rw & -65536, jnp.float32)
                    ca = lax.bitcast_convert_type(cw << 16, jnp.float32)
                    cb = lax.bitcast_convert_type(cw & -65536, jnp.float32)
                    plsc.addupdate(inb[s].at[l, sla], ra + ca)
                    plsc.addupdate(inb[s].at[l, slb], rb + cb)
            return carry

        lax.fori_loop(0, EMBED // (4 * LANES), dim_group, 0)
        pltpu.async_copy(inb[s], out_hbm.at[pl.ds(t0, _CK)], semo[s])

    for i in range(_LA):
        issue(i, i)

    def pipe(c4, carry):
        for s in range(_NS):
            c = c4 * _NS + s
            cn = c + _LA
            sn = (s + _LA) % _NS

            @pl.when(cn < nc)
            def _issue_ahead():
                @pl.when(cn >= _NS)
                def _drain_prev():
                    drain_out(sn)

                issue(cn, sn)

            compute(c, s, s & (LANES // _CK - 1))
        return carry

    lax.fori_loop(0, nc // _NS, pipe, 0)
    for s in range(_NS):
        drain_out(s)


def kernel(input_ids, row_pos_from, row_pos_to, col_pos_from, col_pos_to,
           row_embedding, col_embedding):
    b, n, e = input_ids.shape
    t = b * n
    assert e == EMBED and t % (_NW * _NS * _CK) == 0
    tpw = t // _NW

    x = input_ids.reshape(t, e)
    rpf = row_pos_from.reshape(t)
    rpt = row_pos_to.reshape(t)
    cpf = col_pos_from.reshape(t)
    cpt = col_pos_to.reshape(t)

    def _prep(tab):
        blk = tab.astype(jnp.bfloat16).reshape(DEPTH, e // 32, 2, LANES)
        lo = lax.bitcast_convert_type(blk[:, :, 0, :], jnp.uint16)
        hi = lax.bitcast_convert_type(blk[:, :, 1, :], jnp.uint16)
        w = lo.astype(jnp.uint32) | (hi.astype(jnp.uint32) << 16)
        return lax.bitcast_convert_type(w, jnp.int32).reshape(DEPTH, e // 2)

    tab = jnp.concatenate([_prep(row_embedding), _prep(col_embedding)],
                          axis=0).reshape(-1)

    slot_types = (
        [pltpu.VMEM((_CK, EMBED), jnp.float32) for _ in range(_NS)]
        + [pltpu.SemaphoreType.DMA for _ in range(2 * _NS)]
    )
    mesh = plsc.VectorSubcoreMesh(core_axis_name="c", subcore_axis_name="s")
    run = functools.partial(
        pl.kernel,
        mesh=mesh,
        out_type=jax.ShapeDtypeStruct((t, e), jnp.float32),
        scratch_types=[
            pltpu.VMEM((2 * DEPTH * (EMBED // 2),), jnp.int32),
            pltpu.VMEM((4, tpw), jnp.float32),
            pltpu.VMEM((tpw,), jnp.int32),
            pltpu.VMEM((tpw,), jnp.int32),
            pltpu.SMEM((2 * _CK,), jnp.int32),
        ] + slot_types,
    )(functools.partial(_body, tpw))
    out = run(x, rpf, rpt, cpf, cpt, tab)
    return out.reshape(b, n, e)


# speedup vs baseline: 1.0225x; 1.0225x over previous
import functools

import jax
import jax.numpy as jnp
from jax import lax
from jax.experimental import pallas as pl
from jax.experimental.pallas import tpu as pltpu
from jax.experimental.pallas import tpu_sc as plsc

EMBED = 768
DEPTH = 128
LANES = 16

_NW = 32
_CK = 8
_NS = 8
_LA = 6


def _rne_to_int(x):
    s = x + 0.5
    t = s.astype(jnp.int32)
    d = s - t.astype(jnp.float32)
    nonint = jnp.minimum((d * 16777216.0).astype(jnp.int32), 1)
    return t - ((1 - nonint) & t & 1)


def _mean_idx(f, t):
    s = f + t
    h = s >> 1
    i = h + ((s & h) & 1)
    return jnp.minimum(jnp.maximum(i, 0), DEPTH - 1)


def _body(tpw, in_hbm, rpf_hbm, rpt_hbm, cpf_hbm, cpt_hbm, rtab_hbm,
          ctab_hbm, out_hbm, ctabl, posb, ridx, cbase, sbase, *slotrefs):
    rcb = slotrefs[0:_NS]
    inb = slotrefs[_NS:2 * _NS]
    semg = slotrefs[2 * _NS:3 * _NS]
    semo = slotrefs[3 * _NS:4 * _NS]

    wid = lax.axis_index("s") * 2 + lax.axis_index("c")
    base = wid * tpw
    nc = tpw // _CK

    pltpu.sync_copy(ctab_hbm, ctabl)
    pltpu.sync_copy(rpf_hbm.at[pl.ds(base, tpw)], posb.at[0])
    pltpu.sync_copy(rpt_hbm.at[pl.ds(base, tpw)], posb.at[1])
    pltpu.sync_copy(cpf_hbm.at[pl.ds(base, tpw)], posb.at[2])
    pltpu.sync_copy(cpt_hbm.at[pl.ds(base, tpw)], posb.at[3])

    def idx_body(g, carry):
        sl = pl.ds(g * LANES, LANES)
        rf = _rne_to_int(posb[0, sl] * float(DEPTH))
        rt = _rne_to_int(posb[1, sl] * float(DEPTH))
        cf = _rne_to_int(posb[2, sl] * float(DEPTH))
        ct = _rne_to_int(posb[3, sl] * float(DEPTH))
        ridx[sl] = _mean_idx(rf, rt)
        cbase[sl] = _mean_idx(cf, ct) * (EMBED // 2)
        return carry

    lax.fori_loop(0, tpw // LANES, idx_body, 0)

    def issue(cc, s):
        t0 = base + cc * _CK
        pltpu.async_copy(rtab_hbm.at[ridx.at[pl.ds(cc * _CK, _CK)]],
                         rcb[s], semg[s])
        pltpu.async_copy(in_hbm.at[pl.ds(t0, _CK)], inb[s], semg[s])

    def drain_out(s):
        pltpu.make_async_copy(inb[s], out_hbm.at[pl.ds(base, _CK)],
                              semo[s]).wait()

    def compute(cc, s, sub):
        t0 = base + cc * _CK
        src = in_hbm.at[pl.ds(t0, _CK)]
        pltpu.make_async_copy(rtab_hbm.at[pl.ds(0, _CK)], rcb[s],
                              semg[s]).wait()
        pltpu.make_async_copy(src, inb[s], semg[s]).wait()
        pb = (cc - sub) * _CK
        cvec = cbase[pl.ds(pb, LANES)]
        for l in range(_CK):
            sbase[l] = cvec[sub * _CK + l]

        def dim_group(d, carry):
            doff = d * LANES
            slw = pl.ds(d * LANES, LANES)
            sla = pl.ds(d * 2 * LANES, LANES)
            slb = pl.ds(d * 2 * LANES + LANES, LANES)
            for l in range(_CK):
                rw = rcb[s][l, slw]
                cw = ctabl[pl.ds(sbase[l] + doff, LANES)]
                ra = lax.bitcast_convert_type(rw << 16, jnp.float32)
                rb = lax.bitcast_convert_type(rw & -65536, jnp.float32)
                ca = lax.bitcast_convert_type(cw << 16, jnp.float32)
                cb = lax.bitcast_convert_type(cw & -65536, jnp.float32)
                plsc.addupdate(inb[s].at[l, sla], ra + ca)
                plsc.addupdate(inb[s].at[l, slb], rb + cb)
            return carry

        lax.fori_loop(0, EMBED // (2 * LANES), dim_group, 0)
        pltpu.async_copy(inb[s], out_hbm.at[pl.ds(t0, _CK)], semo[s])

    for i in range(_LA):
        issue(i, i)

    def pipe(c4, carry):
        for s in range(_NS):
            c = c4 * _NS + s
            cn = c + _LA
            sn = (s + _LA) % _NS

            @pl.when(cn < nc)
            def _issue_ahead():
                @pl.when(cn >= _NS)
                def _drain_prev():
                    drain_out(sn)

                issue(cn, sn)

            compute(c, s, s & 1)
        return carry

    lax.fori_loop(0, nc // _NS, pipe, 0)
    for s in range(_NS):
        drain_out(s)


def kernel(input_ids, row_pos_from, row_pos_to, col_pos_from, col_pos_to,
           row_embedding, col_embedding):
    b, n, e = input_ids.shape
    t = b * n
    assert e == EMBED and t % (_NW * _NS * _CK) == 0
    tpw = t // _NW

    x = input_ids.reshape(t, e)
    rpf = row_pos_from.reshape(t)
    rpt = row_pos_to.reshape(t)
    cpf = col_pos_from.reshape(t)
    cpt = col_pos_to.reshape(t)

    def _prep(tab):
        blk = tab.astype(jnp.bfloat16).reshape(DEPTH, e // 32, 2, LANES)
        lo = lax.bitcast_convert_type(blk[:, :, 0, :], jnp.uint16)
        hi = lax.bitcast_convert_type(blk[:, :, 1, :], jnp.uint16)
        w = lo.astype(jnp.uint32) | (hi.astype(jnp.uint32) << 16)
        return lax.bitcast_convert_type(w, jnp.int32).reshape(DEPTH, e // 2)

    rtab = _prep(row_embedding)
    ctab = _prep(col_embedding).reshape(-1)

    slot_types = (
        [pltpu.VMEM((_CK, EMBED // 2), jnp.int32) for _ in range(_NS)]
        + [pltpu.VMEM((_CK, EMBED), jnp.float32) for _ in range(_NS)]
        + [pltpu.SemaphoreType.DMA for _ in range(2 * _NS)]
    )
    mesh = plsc.VectorSubcoreMesh(core_axis_name="c", subcore_axis_name="s")
    run = functools.partial(
        pl.kernel,
        mesh=mesh,
        out_type=jax.ShapeDtypeStruct((t, e), jnp.float32),
        scratch_types=[
            pltpu.VMEM((DEPTH * (EMBED // 2),), jnp.int32),
            pltpu.VMEM((4, tpw), jnp.float32),
            pltpu.VMEM((tpw,), jnp.int32),
            pltpu.VMEM((tpw,), jnp.int32),
            pltpu.SMEM((2 * _CK,), jnp.int32),
        ] + slot_types,
    )(functools.partial(_body, tpw))
    out = run(x, rpf, rpt, cpf, cpt, rtab, ctab)
    return out.reshape(b, n, e)


# speedup vs baseline: 1.0283x; 1.0057x over previous
import functools

import jax
import jax.numpy as jnp
from jax import lax
from jax.experimental import pallas as pl
from jax.experimental.pallas import tpu as pltpu
from jax.experimental.pallas import tpu_sc as plsc

EMBED = 768
DEPTH = 128
LANES = 16

_NW = 32
_CK = 8
_NS = 4
_LA = 2


def _rne_to_int(x):
    s = x + 0.5
    t = s.astype(jnp.int32)
    d = s - t.astype(jnp.float32)
    nonint = jnp.minimum((d * 16777216.0).astype(jnp.int32), 1)
    return t - ((1 - nonint) & t & 1)


def _mean_idx(f, t):
    s = f + t
    h = s >> 1
    i = h + ((s & h) & 1)
    return jnp.minimum(jnp.maximum(i, 0), DEPTH - 1)


def _body(tpw, in_hbm, rpf_hbm, rpt_hbm, cpf_hbm, cpt_hbm, tab_hbm,
          out_hbm, tabl, posb, ridx, cidx, sbase, *slotrefs):
    inb = slotrefs[0:_NS]
    semg = slotrefs[_NS:2 * _NS]
    semo = slotrefs[2 * _NS:3 * _NS]

    wid = lax.axis_index("s") * 2 + lax.axis_index("c")
    base = wid * tpw
    nc = tpw // _CK

    pltpu.sync_copy(tab_hbm, tabl)
    pltpu.sync_copy(rpf_hbm.at[pl.ds(base, tpw)], posb.at[0])
    pltpu.sync_copy(rpt_hbm.at[pl.ds(base, tpw)], posb.at[1])
    pltpu.sync_copy(cpf_hbm.at[pl.ds(base, tpw)], posb.at[2])
    pltpu.sync_copy(cpt_hbm.at[pl.ds(base, tpw)], posb.at[3])

    def idx_body(g, carry):
        sl = pl.ds(g * LANES, LANES)
        rf = _rne_to_int(posb[0, sl] * float(DEPTH))
        rt = _rne_to_int(posb[1, sl] * float(DEPTH))
        cf = _rne_to_int(posb[2, sl] * float(DEPTH))
        ct = _rne_to_int(posb[3, sl] * float(DEPTH))
        ridx[sl] = _mean_idx(rf, rt)
        cidx[sl] = _mean_idx(cf, ct) + DEPTH
        return carry

    lax.fori_loop(0, tpw // LANES, idx_body, 0)

    def issue(cc, s):
        t0 = base + cc * _CK
        pltpu.async_copy(in_hbm.at[pl.ds(t0, _CK)], inb[s], semg[s])

    def drain_out(s):
        pltpu.make_async_copy(inb[s], out_hbm.at[pl.ds(base, _CK)],
                              semo[s]).wait()

    def compute(cc, s, half):
        t0 = base + cc * _CK
        src = in_hbm.at[pl.ds(t0, _CK)]
        pltpu.make_async_copy(src, inb[s], semg[s]).wait()
        pb = (cc - half) * _CK
        rvec = ridx[pl.ds(pb, LANES)] * (EMBED // 2)
        cvec = cidx[pl.ds(pb, LANES)] * (EMBED // 2)
        for l in range(_CK):
            sbase[l] = rvec[half * _CK + l]
            sbase[_CK + l] = cvec[half * _CK + l]

        def dim_group(d2, carry):
            for u in range(2):
                d = d2 * 2 + u
                doff = d * LANES
                sla = pl.ds(d * 2 * LANES, LANES)
                slb = pl.ds(d * 2 * LANES + LANES, LANES)
                for l in range(_CK):
                    rw = tabl[pl.ds(sbase[l] + doff, LANES)]
                    cw = tabl[pl.ds(sbase[_CK + l] + doff, LANES)]
                    ra = lax.bitcast_convert_type(rw << 16, jnp.float32)
                    rb = lax.bitcast_convert_type(rw & -65536, jnp.float32)
                    ca = lax.bitcast_convert_type(cw << 16, jnp.float32)
                    cb = lax.bitcast_convert_type(cw & -65536, jnp.float32)
                    plsc.addupdate(inb[s].at[l, sla], ra + ca)
                    plsc.addupdate(inb[s].at[l, slb], rb + cb)
            return carry

        lax.fori_loop(0, EMBED // (4 * LANES), dim_group, 0)
        pltpu.async_copy(inb[s], out_hbm.at[pl.ds(t0, _CK)], semo[s])

    for i in range(_LA):
        issue(i, i)

    def pipe(c4, carry):
        for s in range(_NS):
            c = c4 * _NS + s
            cn = c + _LA
            sn = (s + _LA) % _NS

            @pl.when(cn < nc)
            def _issue_ahead():
                @pl.when(cn >= _NS)
                def _drain_prev():
                    drain_out(sn)

                issue(cn, sn)

            compute(c, s, s & 1)
        return carry

    lax.fori_loop(0, nc // _NS, pipe, 0)
    for s in range(_NS):
        drain_out(s)


def kernel(input_ids, row_pos_from, row_pos_to, col_pos_from, col_pos_to,
           row_embedding, col_embedding):
    b, n, e = input_ids.shape
    t = b * n
    assert e == EMBED and t % (_NW * _NS * _CK) == 0
    tpw = t // _NW

    x = input_ids.reshape(t, e)
    rpf = row_pos_from.reshape(t)
    rpt = row_pos_to.reshape(t)
    cpf = col_pos_from.reshape(t)
    cpt = col_pos_to.reshape(t)

    def _prep(tab):
        blk = tab.astype(jnp.bfloat16).reshape(DEPTH, e // 32, 2, LANES)
        lo = lax.bitcast_convert_type(blk[:, :, 0, :], jnp.uint16)
        hi = lax.bitcast_convert_type(blk[:, :, 1, :], jnp.uint16)
        w = lo.astype(jnp.uint32) | (hi.astype(jnp.uint32) << 16)
        return lax.bitcast_convert_type(w, jnp.int32).reshape(DEPTH, e // 2)

    tab = jnp.concatenate([_prep(row_embedding), _prep(col_embedding)],
                          axis=0).reshape(-1)

    slot_types = (
        [pltpu.VMEM((_CK, EMBED), jnp.float32) for _ in range(_NS)]
        + [pltpu.SemaphoreType.DMA for _ in range(2 * _NS)]
    )
    mesh = plsc.VectorSubcoreMesh(core_axis_name="c", subcore_axis_name="s")
    run = functools.partial(
        pl.kernel,
        mesh=mesh,
        out_type=jax.ShapeDtypeStruct((t, e), jnp.float32),
        scratch_types=[
            pltpu.VMEM((2 * DEPTH * (EMBED // 2),), jnp.int32),
            pltpu.VMEM((4, tpw), jnp.float32),
            pltpu.VMEM((tpw,), jnp.int32),
            pltpu.VMEM((tpw,), jnp.int32),
            pltpu.SMEM((2 * _CK,), jnp.int32),
        ] + slot_types,
    )(functools.partial(_body, tpw))
    out = run(x, rpf, rpt, cpf, cpt, tab)
    return out.reshape(b, n, e)
